# Initial kernel scaffold; baseline (speedup 1.0000x reference)
import jax
import jax.numpy as jnp
from jax.experimental import pallas as pl


def kernel(h, W, b):
    raise NotImplementedError("write your pallas kernel here")


# speedup vs baseline: 4.7088x; 4.7088x over previous
import functools

import jax
import jax.numpy as jnp
from jax import lax
from jax.experimental import pallas as pl
from jax.experimental.pallas import tpu as pltpu
from jax.experimental.pallas import tpu_sc as plsc

F = 128
NB = 4
N = 100000
KEEP = 50000
NP = 100352
T = 16
C = NP // T
NV = C // 16
RADIX = 1024
KMAX = 0x3FFFFFFF
NBLK = (KEEP + 127) // 128
KPAD = NBLK * 128

SCORE_BLK = 2000


def _score_body(h_ref, w_ref, b_ref, uk_ref):
    hb = h_ref[...]
    acc = lax.dot_general(hb, w_ref[...], (((1,), (0,)), ((), ())))
    x = acc[:, 0] + b_ref[0, 0]
    s = 1.0 / (1.0 + jnp.exp(-x))
    uk = KMAX - lax.bitcast_convert_type(s, jnp.int32)
    uk_ref[...] = uk.reshape(1, 8, SCORE_BLK // 8)


def _score_keys(h2, W, b):
    grid = (h2.shape[0] // SCORE_BLK,)
    wmat = jnp.broadcast_to(W[0][:, None], (F, F))
    uk = pl.pallas_call(
        _score_body,
        grid=grid,
        in_specs=[
            pl.BlockSpec((SCORE_BLK, F), lambda j: (j, 0)),
            pl.BlockSpec((F, F), lambda j: (0, 0)),
            pl.BlockSpec((1, 1), lambda j: (0, 0)),
        ],
        out_specs=pl.BlockSpec((1, 8, SCORE_BLK // 8), lambda j: (j, 0, 0)),
        out_shape=jax.ShapeDtypeStruct((grid[0], 8, SCORE_BLK // 8), jnp.int32),
    )(h2, wmat, b.reshape(1, 1))
    return uk.reshape(NB, N)


def _permute(vec, idx):
    dn = lax.GatherDimensionNumbers(
        offset_dims=(), collapsed_slice_dims=(0,), start_index_map=(0,))
    return lax.gather(vec, idx[:, None], dn, (1,),
                      mode=lax.GatherScatterMode.PROMISE_IN_BOUNDS)


def _sc_sort_gather(ukey_pad, h2):
    mesh = plsc.VectorSubcoreMesh(
        core_axis_name="c", subcore_axis_name="s", num_cores=2, num_subcores=T)

    @functools.partial(
        pl.kernel,
        mesh=mesh,
        compiler_params=pltpu.CompilerParams(needs_layout_passes=False),
        out_type=[
            jax.ShapeDtypeStruct((NB, KEEP, F), jnp.float32),
        ],
        scratch_types=[
            pltpu.VMEM_SHARED((NP,), jnp.int32),
            pltpu.VMEM_SHARED((NP,), jnp.int32),
            pltpu.VMEM_SHARED((NP,), jnp.int32),
            pltpu.VMEM_SHARED((NP,), jnp.int32),
            pltpu.VMEM_SHARED((T, RADIX), jnp.int32),
            pltpu.VMEM((C,), jnp.int32),
            pltpu.VMEM((C,), jnp.int32),
            pltpu.VMEM((C // 128, 128), jnp.int32),
            pltpu.VMEM((RADIX,), jnp.int32),
            pltpu.VMEM((T, RADIX), jnp.int32),
            pltpu.VMEM((16,), jnp.int32),
            pltpu.VMEM((128,), jnp.int32),
            pltpu.VMEM((128,), jnp.int32),
            pltpu.VMEM((128,), jnp.float32),
            pltpu.VMEM((128, F), jnp.float32),
            pltpu.SemaphoreType.DMA,
        ],
    )
    def body(ukey_hbm, h2_hbm, o_hbm,
             A_k, A_v, B_k, B_v, Hgrid,
             ck, cv, dest, cnt, hl, tmp16, idxbuf, keybuf, sbuf, rows, sem):
        core = lax.axis_index("c")
        wid = lax.axis_index("s")
        lane = lax.iota(jnp.int32, 16)
        zeros16 = jnp.zeros((16,), jnp.int32)
        base = wid * C

        def vreg_rank(k, shift):
            d = lax.shift_right_logical(k, shift) & (RADIX - 1)
            key2 = d * 16 + lane
            sk, _ = plsc.sort_key_val(key2, key2)
            sd = lax.shift_right_logical(sk, 4)
            sl = sk & 15
            prev = _permute(sd, jnp.maximum(lane - 1, 0))
            nxt = _permute(sd, jnp.minimum(lane + 1, 15))
            is_new = (lane == 0) | (sd != prev)
            is_last = (lane == 15) | (sd != nxt)
            runstart = plsc.cummax(jnp.where(is_new, lane, zeros16))
            r = lane - runstart
            return sd, sl, r, is_last

        for bi in range(2):
            b = core * 2 + bi
            rowbase = b * N

            for p, shift in enumerate((0, 10, 20)):
                src_k = (None, A_k, B_k)[p]
                src_v = (None, A_v, B_v)[p]
                dst_k = (A_k, B_k, A_k)[p]
                dst_v = (A_v, B_v, A_v)[p]

                if p == 0:
                    pltpu.sync_copy(ukey_hbm.at[b, pl.ds(base, C)], ck)
                else:
                    pltpu.sync_copy(src_k.at[pl.ds(base, C)], ck)
                for j in range(RADIX // 16):
                    cnt[pl.ds(j * 16, 16)] = zeros16

                def hist_body(i, carry):
                    k = ck[pl.ds(i * 16, 16)]
                    sd, _, r, is_last = vreg_rank(k, shift)
                    plsc.addupdate_scatter(cnt, [sd], r + 1, mask=is_last)
                    return carry
                lax.fori_loop(0, NV, hist_body, 0)
                pltpu.sync_copy(cnt, Hgrid.at[wid])
                plsc.subcore_barrier()

                pltpu.sync_copy(Hgrid, hl)

                def off_body(j, carry):
                    tot = zeros16
                    mine = zeros16
                    for w in range(T):
                        hv = hl[w, pl.ds(j * 16, 16)]
                        tot = tot + hv
                        mine = mine + jnp.where(w < wid, hv, zeros16)
                    inc = jnp.cumsum(tot)
                    excl = inc - tot
                    cnt[pl.ds(j * 16, 16)] = excl + mine + carry
                    return carry + lax.reduce_sum(tot, axes=(0,))
                lax.fori_loop(0, RADIX // 16, off_body, jnp.int32(0))

                if p > 0:
                    pltpu.sync_copy(src_v.at[pl.ds(base, C)], cv)

                def rank_body(i, carry):
                    k = ck[pl.ds(i * 16, 16)]
                    sd, sl, r, is_last = vreg_rank(k, shift)
                    off = plsc.load_gather(cnt, [sd])
                    plsc.addupdate_scatter(cnt, [sd], r + 1, mask=is_last)
                    dsort = off + r
                    plsc.store_scatter(tmp16, [sl], dsort)
                    dorig = tmp16[...]
                    row = lax.shift_right_logical(i, 3)
                    col = (i & 7) * 16
                    plsc.store_scatter(
                        dest, [jnp.broadcast_to(row, (16,)), col + lane], dorig)
                    if p == 0:
                        cv[pl.ds(i * 16, 16)] = rowbase + base + i * 16 + lane
                    return carry
                lax.fori_loop(0, NV, rank_body, 0)

                def scat_body(j, carry):
                    pltpu.sync_copy(ck.at[pl.ds(j * 128, 128)],
                                    dst_k.at[dest.at[j]])
                    pltpu.sync_copy(cv.at[pl.ds(j * 128, 128)],
                                    dst_v.at[dest.at[j]])
                    return carry
                lax.fori_loop(0, C // 128, scat_body, 0)
                plsc.subcore_barrier()

            def gath_body(t, carry):
                j = wid + t * T

                @pl.when(j < NBLK)
                def _():
                    pltpu.sync_copy(A_v.at[pl.ds(j * 128, 128)], idxbuf)
                    pltpu.sync_copy(A_k.at[pl.ds(j * 128, 128)], keybuf)
                    pltpu.async_copy(h2_hbm.at[idxbuf], rows, sem).wait()
                    for g in range(8):
                        k16 = keybuf[pl.ds(g * 16, 16)]
                        sbuf[pl.ds(g * 16, 16)] = lax.bitcast_convert_type(
                            KMAX - k16, jnp.float32)

                    def mul_body(r, carry):
                        sv = sbuf[pl.ds((r // 16) * 16, 16)]
                        sb = _permute(sv, jnp.broadcast_to(r % 16, (16,)))
                        for cg in range(8):
                            rows[r, pl.ds(cg * 16, 16)] = (
                                rows[r, pl.ds(cg * 16, 16)] * sb)
                        return carry
                    lax.fori_loop(0, 128, mul_body, 0)

                    @pl.when(j < NBLK - 1)
                    def _():
                        pltpu.sync_copy(rows, o_hbm.at[b, pl.ds(j * 128, 128)])

                    @pl.when(j == NBLK - 1)
                    def _():
                        pltpu.sync_copy(
                            rows.at[pl.ds(0, KEEP - (NBLK - 1) * 128)],
                            o_hbm.at[b, pl.ds((NBLK - 1) * 128,
                                              KEEP - (NBLK - 1) * 128)])
                return carry
            lax.fori_loop(0, (NBLK + T - 1) // T, gath_body, 0)
            plsc.subcore_barrier()

    return body(ukey_pad, h2)


def kernel(h, W, b):
    h2 = h.reshape(NB * N, F)
    ukey = _score_keys(h2, W, b)
    ukey_pad = jnp.concatenate(
        [ukey, jnp.full((NB, NP - N), KMAX, jnp.int32)], axis=1)
    (out,) = _sc_sort_gather(ukey_pad, h2)
    return out


# speedup vs baseline: 5.6348x; 1.1966x over previous
import functools

import jax
import jax.numpy as jnp
from jax import lax
from jax.experimental import pallas as pl
from jax.experimental.pallas import tpu as pltpu
from jax.experimental.pallas import tpu_sc as plsc

F = 128
NB = 4
N = 100000
KEEP = 50000
NP = 100352
T = 16
C = NP // T
NV = C // 16
RADIX = 1024
KMAX = 0x3FFFFFFF
NBLK = (KEEP + 127) // 128
KPAD = NBLK * 128

SCORE_BLK = 2000


def _score_body(h_ref, w_ref, b_ref, uk_ref):
    hb = h_ref[...]
    acc = lax.dot_general(hb, w_ref[...], (((1,), (0,)), ((), ())))
    x = acc[:, 0] + b_ref[0, 0]
    s = 1.0 / (1.0 + jnp.exp(-x))
    uk = KMAX - lax.bitcast_convert_type(s, jnp.int32)
    uk_ref[...] = uk.reshape(1, 8, SCORE_BLK // 8)


def _score_keys(h2, W, b):
    grid = (h2.shape[0] // SCORE_BLK,)
    wmat = jnp.broadcast_to(W[0][:, None], (F, F))
    uk = pl.pallas_call(
        _score_body,
        grid=grid,
        in_specs=[
            pl.BlockSpec((SCORE_BLK, F), lambda j: (j, 0)),
            pl.BlockSpec((F, F), lambda j: (0, 0)),
            pl.BlockSpec((1, 1), lambda j: (0, 0)),
        ],
        out_specs=pl.BlockSpec((1, 8, SCORE_BLK // 8), lambda j: (j, 0, 0)),
        out_shape=jax.ShapeDtypeStruct((grid[0], 8, SCORE_BLK // 8), jnp.int32),
    )(h2, wmat, b.reshape(1, 1))
    return uk.reshape(NB, N)


def _permute(vec, idx):
    dn = lax.GatherDimensionNumbers(
        offset_dims=(), collapsed_slice_dims=(0,), start_index_map=(0,))
    return lax.gather(vec, idx[:, None], dn, (1,),
                      mode=lax.GatherScatterMode.PROMISE_IN_BOUNDS)


def _sc_sort_gather(ukey_pad, h2):
    mesh = plsc.VectorSubcoreMesh(
        core_axis_name="c", subcore_axis_name="s", num_cores=2, num_subcores=T)

    @functools.partial(
        pl.kernel,
        mesh=mesh,
        compiler_params=pltpu.CompilerParams(needs_layout_passes=False),
        out_type=[
            jax.ShapeDtypeStruct((NB, KEEP, F), jnp.float32),
        ],
        scratch_types=[
            pltpu.VMEM_SHARED((NP,), jnp.int32),
            pltpu.VMEM_SHARED((NP,), jnp.int32),
            pltpu.VMEM_SHARED((NP,), jnp.int32),
            pltpu.VMEM_SHARED((NP,), jnp.int32),
            pltpu.VMEM_SHARED((T, RADIX), jnp.int32),
            pltpu.VMEM((C,), jnp.int32),
            pltpu.VMEM((C,), jnp.int32),
            pltpu.VMEM((C // 128, 128), jnp.int32),
            pltpu.VMEM((RADIX,), jnp.int32),
            pltpu.VMEM((T, RADIX), jnp.int32),
            pltpu.VMEM((16,), jnp.int32),
            pltpu.VMEM((2, 128), jnp.int32),
            pltpu.VMEM((2, 128), jnp.int32),
            pltpu.VMEM((128,), jnp.float32),
            pltpu.VMEM((2, 128, F), jnp.float32),
            pltpu.SemaphoreType.DMA,
            pltpu.SemaphoreType.DMA,
            pltpu.SemaphoreType.DMA,
        ],
    )
    def body(ukey_hbm, h2_hbm, o_hbm,
             A_k, A_v, B_k, B_v, Hgrid,
             ck, cv, dest, cnt, hl, tmp16, idxbuf, keybuf, sbuf, rows,
             sem, gsem0, gsem1):
        core = lax.axis_index("c")
        wid = lax.axis_index("s")
        lane = lax.iota(jnp.int32, 16)
        zeros16 = jnp.zeros((16,), jnp.int32)
        base = wid * C

        def vreg_rank(k, shift):
            d = lax.shift_right_logical(k, shift) & (RADIX - 1)
            key2 = d * 16 + lane
            sk, _ = plsc.sort_key_val(key2, key2)
            sd = lax.shift_right_logical(sk, 4)
            sl = sk & 15
            prev = _permute(sd, jnp.maximum(lane - 1, 0))
            nxt = _permute(sd, jnp.minimum(lane + 1, 15))
            is_new = (lane == 0) | (sd != prev)
            is_last = (lane == 15) | (sd != nxt)
            runstart = plsc.cummax(jnp.where(is_new, lane, zeros16))
            r = lane - runstart
            return sd, sl, r, is_last

        for bi in range(2):
            b = core * 2 + bi
            rowbase = b * N

            for p, shift in enumerate((0, 10, 20)):
                src_k = (None, A_k, B_k)[p]
                src_v = (None, A_v, B_v)[p]
                dst_k = (A_k, B_k, A_k)[p]
                dst_v = (A_v, B_v, A_v)[p]

                if p == 0:
                    pltpu.sync_copy(ukey_hbm.at[b, pl.ds(base, C)], ck)
                else:
                    pltpu.sync_copy(src_k.at[pl.ds(base, C)], ck)
                for j in range(RADIX // 16):
                    cnt[pl.ds(j * 16, 16)] = zeros16

                def hist_body(i, carry):
                    k = ck[pl.ds(i * 16, 16)]
                    sd, _, r, is_last = vreg_rank(k, shift)
                    plsc.addupdate_scatter(cnt, [sd], r + 1, mask=is_last)
                    return carry
                lax.fori_loop(0, NV, hist_body, 0)
                pltpu.sync_copy(cnt, Hgrid.at[wid])
                plsc.subcore_barrier()

                pltpu.sync_copy(Hgrid, hl)

                def off_body(j, carry):
                    tot = zeros16
                    mine = zeros16
                    for w in range(T):
                        hv = hl[w, pl.ds(j * 16, 16)]
                        tot = tot + hv
                        mine = mine + jnp.where(w < wid, hv, zeros16)
                    inc = jnp.cumsum(tot)
                    excl = inc - tot
                    cnt[pl.ds(j * 16, 16)] = excl + mine + carry
                    return carry + lax.reduce_sum(tot, axes=(0,))
                lax.fori_loop(0, RADIX // 16, off_body, jnp.int32(0))

                if p > 0:
                    pltpu.sync_copy(src_v.at[pl.ds(base, C)], cv)

                def rank_body(i, carry):
                    k = ck[pl.ds(i * 16, 16)]
                    sd, sl, r, is_last = vreg_rank(k, shift)
                    off = plsc.load_gather(cnt, [sd])
                    plsc.addupdate_scatter(cnt, [sd], r + 1, mask=is_last)
                    dsort = off + r
                    plsc.store_scatter(tmp16, [sl], dsort)
                    dorig = tmp16[...]
                    row = lax.shift_right_logical(i, 3)
                    col = (i & 7) * 16
                    plsc.store_scatter(
                        dest, [jnp.broadcast_to(row, (16,)), col + lane], dorig)
                    if p == 0:
                        cv[pl.ds(i * 16, 16)] = rowbase + base + i * 16 + lane
                    return carry
                lax.fori_loop(0, NV, rank_body, 0)

                def scat_body(jg, carry):
                    handles = []
                    for u in range(7):
                        j = jg * 7 + u
                        handles.append(pltpu.async_copy(
                            ck.at[pl.ds(j * 128, 128)],
                            dst_k.at[dest.at[j]], sem))
                        handles.append(pltpu.async_copy(
                            cv.at[pl.ds(j * 128, 128)],
                            dst_v.at[dest.at[j]], sem))
                    for hdl in handles:
                        hdl.wait()
                    return carry
                lax.fori_loop(0, C // 128 // 7, scat_body, 0)
                plsc.subcore_barrier()

            def _start(u, j):
                iv, kv, rv = idxbuf.at[u], keybuf.at[u], rows.at[u]
                gs = (gsem0, gsem1)[u]
                pltpu.sync_copy(A_v.at[pl.ds(j * 128, 128)], iv)
                pltpu.sync_copy(A_k.at[pl.ds(j * 128, 128)], kv)
                pltpu.async_copy(h2_hbm.at[iv], rv, gs)

            def _consume(u, j):
                iv, kv, rv = idxbuf.at[u], keybuf.at[u], rows.at[u]
                gs = (gsem0, gsem1)[u]
                pltpu.make_async_copy(h2_hbm.at[iv], rv, gs).wait()
                for g in range(8):
                    k16 = kv[pl.ds(g * 16, 16)]
                    sbuf[pl.ds(g * 16, 16)] = lax.bitcast_convert_type(
                        KMAX - k16, jnp.float32)

                def mul_body(r, carry):
                    sv = sbuf[pl.ds((r // 16) * 16, 16)]
                    sb = _permute(sv, jnp.broadcast_to(r % 16, (16,)))
                    for cg in range(8):
                        rv[r, pl.ds(cg * 16, 16)] = (
                            rv[r, pl.ds(cg * 16, 16)] * sb)
                    return carry
                lax.fori_loop(0, 128, mul_body, 0)

                @pl.when(j < NBLK - 1)
                def _():
                    pltpu.sync_copy(rv, o_hbm.at[b, pl.ds(j * 128, 128)])

                @pl.when(j == NBLK - 1)
                def _():
                    pltpu.sync_copy(
                        rv.at[pl.ds(0, KEEP - (NBLK - 1) * 128)],
                        o_hbm.at[b, pl.ds((NBLK - 1) * 128,
                                          KEEP - (NBLK - 1) * 128)])

            def gath_body(t, carry):
                j = wid + t * T
                jprev = j - T
                even = (t & 1) == 0

                @pl.when(j < NBLK)
                def _():
                    @pl.when(even)
                    def _():
                        _start(0, j)

                    @pl.when(jnp.logical_not(even))
                    def _():
                        _start(1, j)

                @pl.when((t >= 1) & (jprev < NBLK))
                def _():
                    @pl.when(even)
                    def _():
                        _consume(1, jprev)

                    @pl.when(jnp.logical_not(even))
                    def _():
                        _consume(0, jprev)
                return carry
            lax.fori_loop(0, (NBLK + T - 1) // T + 1, gath_body, 0)
            plsc.subcore_barrier()

    return body(ukey_pad, h2)


def kernel(h, W, b):
    h2 = h.reshape(NB * N, F)
    ukey = _score_keys(h2, W, b)
    ukey_pad = jnp.concatenate(
        [ukey, jnp.full((NB, NP - N), KMAX, jnp.int32)], axis=1)
    (out,) = _sc_sort_gather(ukey_pad, h2)
    return out


# speedup vs baseline: 6.2894x; 1.1162x over previous
import functools

import jax
import jax.numpy as jnp
from jax import lax
from jax.experimental import pallas as pl
from jax.experimental.pallas import tpu as pltpu
from jax.experimental.pallas import tpu_sc as plsc

F = 128
NB = 4
N = 100000
KEEP = 50000
NP = 100352
T = 16
C = NP // T
NV = C // 16
RADIX = 1024
KMAX = 0x3FFFFFFF
NBLK = (KEEP + 127) // 128
KPAD = NBLK * 128

SCORE_BLK = 2000


def _score_body(h_ref, w_ref, b_ref, uk_ref):
    hb = h_ref[...]
    acc = lax.dot_general(hb, w_ref[...], (((1,), (0,)), ((), ())))
    x = acc[:, 0] + b_ref[0, 0]
    s = 1.0 / (1.0 + jnp.exp(-x))
    uk = KMAX - lax.bitcast_convert_type(s, jnp.int32)
    uk_ref[...] = uk.reshape(1, 8, SCORE_BLK // 8)


def _score_keys(h2, W, b):
    grid = (h2.shape[0] // SCORE_BLK,)
    wmat = jnp.broadcast_to(W[0][:, None], (F, F))
    uk = pl.pallas_call(
        _score_body,
        grid=grid,
        in_specs=[
            pl.BlockSpec((SCORE_BLK, F), lambda j: (j, 0)),
            pl.BlockSpec((F, F), lambda j: (0, 0)),
            pl.BlockSpec((1, 1), lambda j: (0, 0)),
        ],
        out_specs=pl.BlockSpec((1, 8, SCORE_BLK // 8), lambda j: (j, 0, 0)),
        out_shape=jax.ShapeDtypeStruct((grid[0], 8, SCORE_BLK // 8), jnp.int32),
    )(h2, wmat, b.reshape(1, 1))
    return uk.reshape(NB, N)


def _permute(vec, idx):
    dn = lax.GatherDimensionNumbers(
        offset_dims=(), collapsed_slice_dims=(0,), start_index_map=(0,))
    return lax.gather(vec, idx[:, None], dn, (1,),
                      mode=lax.GatherScatterMode.PROMISE_IN_BOUNDS)


def _sc_sort_gather(ukey_pad, h2):
    mesh = plsc.VectorSubcoreMesh(
        core_axis_name="c", subcore_axis_name="s", num_cores=2, num_subcores=T)

    @functools.partial(
        pl.kernel,
        mesh=mesh,
        compiler_params=pltpu.CompilerParams(needs_layout_passes=False),
        out_type=[
            jax.ShapeDtypeStruct((NB, KEEP, F), jnp.float32),
        ],
        scratch_types=[
            pltpu.VMEM_SHARED((NP,), jnp.int32),
            pltpu.VMEM_SHARED((NP,), jnp.int32),
            pltpu.VMEM_SHARED((NP,), jnp.int32),
            pltpu.VMEM_SHARED((NP,), jnp.int32),
            pltpu.VMEM_SHARED((T, RADIX), jnp.int32),
            pltpu.VMEM((C,), jnp.int32),
            pltpu.VMEM((C,), jnp.int32),
            pltpu.VMEM((C,), jnp.int32),
            pltpu.VMEM((C // 128, 128), jnp.int32),
            pltpu.VMEM((RADIX,), jnp.int32),
            pltpu.VMEM((T, RADIX), jnp.int32),
            pltpu.VMEM((16,), jnp.int32),
            pltpu.VMEM((2, 128), jnp.int32),
            pltpu.VMEM((2, 128), jnp.int32),
            pltpu.VMEM((128,), jnp.float32),
            pltpu.VMEM((2, 128, F), jnp.float32),
            pltpu.SemaphoreType.DMA,
            pltpu.SemaphoreType.DMA,
            pltpu.SemaphoreType.DMA,
        ],
    )
    def body(ukey_hbm, h2_hbm, o_hbm,
             A_k, A_v, B_k, B_v, Hgrid,
             ck, cv, rinfo, dest, cnt, hl, tmp16, idxbuf, keybuf, sbuf, rows,
             sem, gsem0, gsem1):
        core = lax.axis_index("c")
        wid = lax.axis_index("s")
        lane = lax.iota(jnp.int32, 16)
        zeros16 = jnp.zeros((16,), jnp.int32)
        base = wid * C

        def vreg_rank(k, shift):
            d = lax.shift_right_logical(k, shift) & (RADIX - 1)
            key2 = d * 16 + lane
            sk, _ = plsc.sort_key_val(key2, key2)
            sd = lax.shift_right_logical(sk, 4)
            sl = sk & 15
            prev = _permute(sd, jnp.maximum(lane - 1, 0))
            nxt = _permute(sd, jnp.minimum(lane + 1, 15))
            is_new = (lane == 0) | (sd != prev)
            is_last = (lane == 15) | (sd != nxt)
            runstart = plsc.cummax(jnp.where(is_new, lane, zeros16))
            r = lane - runstart
            return sd, sl, r, is_last

        for bi in range(2):
            b = core * 2 + bi
            rowbase = b * N

            for p, shift in enumerate((0, 10, 20)):
                src_k = (None, A_k, B_k)[p]
                src_v = (None, A_v, B_v)[p]
                dst_k = (A_k, B_k, A_k)[p]
                dst_v = (A_v, B_v, A_v)[p]

                if p == 0:
                    pltpu.sync_copy(ukey_hbm.at[b, pl.ds(base, C)], ck)
                else:
                    pltpu.sync_copy(src_k.at[pl.ds(base, C)], ck)
                for j in range(RADIX // 16):
                    cnt[pl.ds(j * 16, 16)] = zeros16

                def hist_body(i, carry):
                    k = ck[pl.ds(i * 16, 16)]
                    sd, sl, r, is_last = vreg_rank(k, shift)
                    plsc.addupdate_scatter(cnt, [sd], r + 1, mask=is_last)
                    rinfo[pl.ds(i * 16, 16)] = (
                        sd | (sl << 10) | (r << 14)
                        | (is_last.astype(jnp.int32) << 18))
                    return carry
                lax.fori_loop(0, NV, hist_body, 0)
                pltpu.sync_copy(cnt, Hgrid.at[wid])
                plsc.subcore_barrier()

                pltpu.sync_copy(Hgrid, hl)

                def off_body(j, carry):
                    tot = zeros16
                    mine = zeros16
                    for w in range(T):
                        hv = hl[w, pl.ds(j * 16, 16)]
                        tot = tot + hv
                        mine = mine + jnp.where(w < wid, hv, zeros16)
                    inc = jnp.cumsum(tot)
                    excl = inc - tot
                    cnt[pl.ds(j * 16, 16)] = excl + mine + carry
                    return carry + lax.reduce_sum(tot, axes=(0,))
                lax.fori_loop(0, RADIX // 16, off_body, jnp.int32(0))

                if p > 0:
                    pltpu.sync_copy(src_v.at[pl.ds(base, C)], cv)

                def rank_body(i, carry):
                    pk = rinfo[pl.ds(i * 16, 16)]
                    sd = pk & (RADIX - 1)
                    sl = lax.shift_right_logical(pk, 10) & 15
                    r = lax.shift_right_logical(pk, 14) & 15
                    is_last = (lax.shift_right_logical(pk, 18) & 1) == 1
                    off = plsc.load_gather(cnt, [sd])
                    plsc.addupdate_scatter(cnt, [sd], r + 1, mask=is_last)
                    dsort = off + r
                    plsc.store_scatter(tmp16, [sl], dsort)
                    dorig = tmp16[...]
                    row = lax.shift_right_logical(i, 3)
                    col = (i & 7) * 16
                    plsc.store_scatter(
                        dest, [jnp.broadcast_to(row, (16,)), col + lane], dorig)
                    if p == 0:
                        cv[pl.ds(i * 16, 16)] = rowbase + base + i * 16 + lane
                    return carry
                lax.fori_loop(0, NV, rank_body, 0)

                def scat_body(jg, carry):
                    handles = []
                    for u in range(7):
                        j = jg * 7 + u
                        handles.append(pltpu.async_copy(
                            ck.at[pl.ds(j * 128, 128)],
                            dst_k.at[dest.at[j]], sem))
                        handles.append(pltpu.async_copy(
                            cv.at[pl.ds(j * 128, 128)],
                            dst_v.at[dest.at[j]], sem))
                    for hdl in handles:
                        hdl.wait()
                    return carry
                lax.fori_loop(0, C // 128 // 7, scat_body, 0)
                plsc.subcore_barrier()

            def _start(u, j):
                iv, kv, rv = idxbuf.at[u], keybuf.at[u], rows.at[u]
                gs = (gsem0, gsem1)[u]
                pltpu.sync_copy(A_v.at[pl.ds(j * 128, 128)], iv)
                pltpu.sync_copy(A_k.at[pl.ds(j * 128, 128)], kv)
                pltpu.async_copy(h2_hbm.at[iv], rv, gs)

            def _consume(u, j):
                iv, kv, rv = idxbuf.at[u], keybuf.at[u], rows.at[u]
                gs = (gsem0, gsem1)[u]
                pltpu.make_async_copy(h2_hbm.at[iv], rv, gs).wait()
                for g in range(8):
                    k16 = kv[pl.ds(g * 16, 16)]
                    sbuf[pl.ds(g * 16, 16)] = lax.bitcast_convert_type(
                        KMAX - k16, jnp.float32)

                def mul_body(t4, carry):
                    r0 = t4 * 4
                    sv = sbuf[pl.ds((r0 // 16) * 16, 16)]
                    for u in range(4):
                        r = r0 + u
                        sb = _permute(sv, jnp.broadcast_to(r % 16, (16,)))
                        for cg in range(8):
                            rv[r, pl.ds(cg * 16, 16)] = (
                                rv[r, pl.ds(cg * 16, 16)] * sb)
                    return carry
                lax.fori_loop(0, 32, mul_body, 0)

                @pl.when(j < NBLK - 1)
                def _():
                    pltpu.sync_copy(rv, o_hbm.at[b, pl.ds(j * 128, 128)])

                @pl.when(j == NBLK - 1)
                def _():
                    pltpu.sync_copy(
                        rv.at[pl.ds(0, KEEP - (NBLK - 1) * 128)],
                        o_hbm.at[b, pl.ds((NBLK - 1) * 128,
                                          KEEP - (NBLK - 1) * 128)])

            def gath_body(t, carry):
                j = wid + t * T
                jprev = j - T
                even = (t & 1) == 0

                @pl.when(j < NBLK)
                def _():
                    @pl.when(even)
                    def _():
                        _start(0, j)

                    @pl.when(jnp.logical_not(even))
                    def _():
                        _start(1, j)

                @pl.when((t >= 1) & (jprev < NBLK))
                def _():
                    @pl.when(even)
                    def _():
                        _consume(1, jprev)

                    @pl.when(jnp.logical_not(even))
                    def _():
                        _consume(0, jprev)
                return carry
            lax.fori_loop(0, (NBLK + T - 1) // T + 1, gath_body, 0)
            plsc.subcore_barrier()

    return body(ukey_pad, h2)


def kernel(h, W, b):
    h2 = h.reshape(NB * N, F)
    ukey = _score_keys(h2, W, b)
    ukey_pad = jnp.concatenate(
        [ukey, jnp.full((NB, NP - N), KMAX, jnp.int32)], axis=1)
    (out,) = _sc_sort_gather(ukey_pad, h2)
    return out


# speedup vs baseline: 7.8178x; 1.2430x over previous
import functools

import jax
import jax.numpy as jnp
from jax import lax
from jax.experimental import pallas as pl
from jax.experimental.pallas import tpu as pltpu
from jax.experimental.pallas import tpu_sc as plsc

F = 128
NB = 4
N = 100000
KEEP = 50000
NP = 100352
T = 16
C = NP // T
NV = C // 16
RADIX = 1024
KMAX = 0x3FFFFFFF
NBLK = (KEEP + 127) // 128
KPAD = NBLK * 128

SCORE_BLK = 4000


def _score_body(h_ref, w_ref, b_ref, uk_ref):
    hb = h_ref[...]
    acc = lax.dot_general(hb, w_ref[...], (((1,), (0,)), ((), ())))
    x = acc[:, 0] + b_ref[0, 0]
    s = 1.0 / (1.0 + jnp.exp(-x))
    uk = KMAX - lax.bitcast_convert_type(s, jnp.int32)
    uk_ref[...] = uk.reshape(1, 8, SCORE_BLK // 8)


def _score_keys(h2, W, b):
    grid = (h2.shape[0] // SCORE_BLK,)
    wmat = jnp.broadcast_to(W[0][:, None], (F, F))
    uk = pl.pallas_call(
        _score_body,
        grid=grid,
        in_specs=[
            pl.BlockSpec((SCORE_BLK, F), lambda j: (j, 0)),
            pl.BlockSpec((F, F), lambda j: (0, 0)),
            pl.BlockSpec((1, 1), lambda j: (0, 0)),
        ],
        out_specs=pl.BlockSpec((1, 8, SCORE_BLK // 8), lambda j: (j, 0, 0)),
        out_shape=jax.ShapeDtypeStruct((grid[0], 8, SCORE_BLK // 8), jnp.int32),
    )(h2, wmat, b.reshape(1, 1))
    return uk.reshape(NB, N)


def _permute(vec, idx):
    dn = lax.GatherDimensionNumbers(
        offset_dims=(), collapsed_slice_dims=(0,), start_index_map=(0,))
    return lax.gather(vec, idx[:, None], dn, (1,),
                      mode=lax.GatherScatterMode.PROMISE_IN_BOUNDS)


def _sc_sort_gather(ukey_pad, h2):
    mesh = plsc.VectorSubcoreMesh(
        core_axis_name="c", subcore_axis_name="s", num_cores=2, num_subcores=T)

    @functools.partial(
        pl.kernel,
        mesh=mesh,
        compiler_params=pltpu.CompilerParams(needs_layout_passes=False),
        out_type=[
            jax.ShapeDtypeStruct((NB, KEEP, F), jnp.float32),
        ],
        scratch_types=[
            pltpu.VMEM_SHARED((NP,), jnp.int32),
            pltpu.VMEM_SHARED((NP,), jnp.int32),
            pltpu.VMEM_SHARED((NP,), jnp.int32),
            pltpu.VMEM_SHARED((NP,), jnp.int32),
            pltpu.VMEM_SHARED((T, RADIX), jnp.int32),
            pltpu.VMEM((C,), jnp.int32),
            pltpu.VMEM((C,), jnp.int32),
            pltpu.VMEM((C,), jnp.int32),
            pltpu.VMEM((C // 128, 128), jnp.int32),
            pltpu.VMEM((RADIX,), jnp.int32),
            pltpu.VMEM((T, RADIX), jnp.int32),
            pltpu.VMEM((16,), jnp.int32),
            pltpu.VMEM((2, 128), jnp.int32),
            pltpu.VMEM((2, 128), jnp.int32),
            pltpu.VMEM((128,), jnp.float32),
            pltpu.VMEM((2, 128, F), jnp.float32),
            pltpu.SemaphoreType.DMA,
            pltpu.SemaphoreType.DMA,
            pltpu.SemaphoreType.DMA,
            pltpu.SemaphoreType.DMA,
            pltpu.SemaphoreType.DMA,
        ],
    )
    def body(ukey_hbm, h2_hbm, o_hbm,
             A_k, A_v, B_k, B_v, Hgrid,
             ck, cv, rinfo, dest, cnt, hl, tmp16, idxbuf, keybuf, sbuf, rows,
             sem, gsem0, gsem1, wsem0, wsem1):
        core = lax.axis_index("c")
        wid = lax.axis_index("s")
        lane = lax.iota(jnp.int32, 16)
        zeros16 = jnp.zeros((16,), jnp.int32)
        base = wid * C

        def vreg_rank(k, shift):
            d = lax.shift_right_logical(k, shift) & (RADIX - 1)
            key2 = d * 16 + lane
            sk, _ = plsc.sort_key_val(key2, key2)
            sd = lax.shift_right_logical(sk, 4)
            sl = sk & 15
            prev = _permute(sd, jnp.maximum(lane - 1, 0))
            nxt = _permute(sd, jnp.minimum(lane + 1, 15))
            is_new = (lane == 0) | (sd != prev)
            is_last = (lane == 15) | (sd != nxt)
            runstart = plsc.cummax(jnp.where(is_new, lane, zeros16))
            r = lane - runstart
            return sd, sl, r, is_last

        for bi in range(2):
            b = core * 2 + bi
            rowbase = b * N

            for p, shift in enumerate((0, 10, 20)):
                src_k = (None, A_k, B_k)[p]
                src_v = (None, A_v, B_v)[p]
                dst_k = (A_k, B_k, A_k)[p]
                dst_v = (A_v, B_v, A_v)[p]

                if p == 0:
                    pltpu.sync_copy(ukey_hbm.at[b, pl.ds(base, C)], ck)
                else:
                    pltpu.sync_copy(src_k.at[pl.ds(base, C)], ck)
                for j in range(RADIX // 16):
                    cnt[pl.ds(j * 16, 16)] = zeros16

                def hist_body(i, carry):
                    k = ck[pl.ds(i * 16, 16)]
                    sd, sl, r, is_last = vreg_rank(k, shift)
                    plsc.addupdate_scatter(cnt, [sd], r + 1, mask=is_last)
                    rinfo[pl.ds(i * 16, 16)] = (
                        sd | (sl << 10) | (r << 14)
                        | (is_last.astype(jnp.int32) << 18))
                    return carry
                lax.fori_loop(0, NV, hist_body, 0)
                pltpu.sync_copy(cnt, Hgrid.at[wid])
                plsc.subcore_barrier()

                pltpu.sync_copy(Hgrid, hl)

                def off_body(j, carry):
                    tot = zeros16
                    mine = zeros16
                    for w in range(T):
                        hv = hl[w, pl.ds(j * 16, 16)]
                        tot = tot + hv
                        mine = mine + jnp.where(w < wid, hv, zeros16)
                    inc = jnp.cumsum(tot)
                    excl = inc - tot
                    cnt[pl.ds(j * 16, 16)] = excl + mine + carry
                    return carry + lax.reduce_sum(tot, axes=(0,))
                lax.fori_loop(0, RADIX // 16, off_body, jnp.int32(0))

                if p > 0:
                    pltpu.sync_copy(src_v.at[pl.ds(base, C)], cv)

                def rank_body(i, carry):
                    pk = rinfo[pl.ds(i * 16, 16)]
                    sd = pk & (RADIX - 1)
                    sl = lax.shift_right_logical(pk, 10) & 15
                    r = lax.shift_right_logical(pk, 14) & 15
                    is_last = (lax.shift_right_logical(pk, 18) & 1) == 1
                    off = plsc.load_gather(cnt, [sd])
                    plsc.addupdate_scatter(cnt, [sd], r + 1, mask=is_last)
                    dsort = off + r
                    plsc.store_scatter(tmp16, [sl], dsort)
                    dorig = tmp16[...]
                    row = lax.shift_right_logical(i, 3)
                    col = (i & 7) * 16
                    plsc.store_scatter(
                        dest, [jnp.broadcast_to(row, (16,)), col + lane], dorig)
                    if p == 0:
                        cv[pl.ds(i * 16, 16)] = rowbase + base + i * 16 + lane
                    return carry
                lax.fori_loop(0, NV, rank_body, 0)

                def scat_body(jg, carry):
                    handles = []
                    for u in range(7):
                        j = jg * 7 + u
                        handles.append(pltpu.async_copy(
                            ck.at[pl.ds(j * 128, 128)],
                            dst_k.at[dest.at[j]], sem))
                        handles.append(pltpu.async_copy(
                            cv.at[pl.ds(j * 128, 128)],
                            dst_v.at[dest.at[j]], sem))
                    for hdl in handles:
                        hdl.wait()
                    return carry
                lax.fori_loop(0, C // 128 // 7, scat_body, 0)
                plsc.subcore_barrier()

            def _start(u, j):
                iv, kv, rv = idxbuf.at[u], keybuf.at[u], rows.at[u]
                gs = (gsem0, gsem1)[u]
                pltpu.sync_copy(A_v.at[pl.ds(j * 128, 128)], iv)
                pltpu.sync_copy(A_k.at[pl.ds(j * 128, 128)], kv)
                pltpu.async_copy(h2_hbm.at[iv], rv, gs)

            def _consume(u, j):
                iv, kv, rv = idxbuf.at[u], keybuf.at[u], rows.at[u]
                gs = (gsem0, gsem1)[u]
                ws = (wsem0, wsem1)[u]
                pltpu.make_async_copy(h2_hbm.at[iv], rv, gs).wait()
                for g in range(8):
                    k16 = kv[pl.ds(g * 16, 16)]
                    sbuf[pl.ds(g * 16, 16)] = lax.bitcast_convert_type(
                        KMAX - k16, jnp.float32)

                def mul_body(t4, carry):
                    r0 = t4 * 4
                    sv = sbuf[pl.ds((r0 // 16) * 16, 16)]
                    for u in range(4):
                        r = r0 + u
                        sb = _permute(sv, jnp.broadcast_to(r % 16, (16,)))
                        for cg in range(8):
                            rv[r, pl.ds(cg * 16, 16)] = (
                                rv[r, pl.ds(cg * 16, 16)] * sb)
                    return carry
                lax.fori_loop(0, 32, mul_body, 0)

                @pl.when(j >= 2 * T)
                def _():
                    pltpu.make_async_copy(
                        rv, o_hbm.at[b, pl.ds((j - 2 * T) * 128, 128)],
                        ws).wait()

                @pl.when(j < NBLK - 1)
                def _():
                    pltpu.async_copy(rv, o_hbm.at[b, pl.ds(j * 128, 128)], ws)

                @pl.when(j == NBLK - 1)
                def _():
                    pltpu.sync_copy(
                        rv.at[pl.ds(0, KEEP - (NBLK - 1) * 128)],
                        o_hbm.at[b, pl.ds((NBLK - 1) * 128,
                                          KEEP - (NBLK - 1) * 128)])

            def gath_body(t, carry):
                j = wid + t * T
                jprev = j - T
                even = (t & 1) == 0

                @pl.when(j < NBLK)
                def _():
                    @pl.when(even)
                    def _():
                        _start(0, j)

                    @pl.when(jnp.logical_not(even))
                    def _():
                        _start(1, j)

                @pl.when((t >= 1) & (jprev < NBLK))
                def _():
                    @pl.when(even)
                    def _():
                        _consume(1, jprev)

                    @pl.when(jnp.logical_not(even))
                    def _():
                        _consume(0, jprev)
                return carry
            lax.fori_loop(0, (NBLK + T - 1) // T + 1, gath_body, 0)
            for u in range(2):
                m = wid + u * T
                jlast = m + ((NBLK - 1 - m) // (2 * T)) * (2 * T)

                @pl.when(jlast != NBLK - 1)
                def _():
                    pltpu.make_async_copy(
                        rows.at[u], o_hbm.at[b, pl.ds(jlast * 128, 128)],
                        (wsem0, wsem1)[u]).wait()
            plsc.subcore_barrier()

    return body(ukey_pad, h2)


def kernel(h, W, b):
    h2 = h.reshape(NB * N, F)
    ukey = _score_keys(h2, W, b)
    ukey_pad = jnp.concatenate(
        [ukey, jnp.full((NB, NP - N), KMAX, jnp.int32)], axis=1)
    (out,) = _sc_sort_gather(ukey_pad, h2)
    return out


# speedup vs baseline: 8.6759x; 1.1098x over previous
import functools

import jax
import jax.numpy as jnp
from jax import lax
from jax.experimental import pallas as pl
from jax.experimental.pallas import tpu as pltpu
from jax.experimental.pallas import tpu_sc as plsc

F = 128
NB = 4
N = 100000
KEEP = 50000
NP = 100352
T = 16
C = NP // T
NV = C // 16
RADIX = 1024
KMAX = 0x3FFFFFFF
NBLK = (KEEP + 127) // 128
KPAD = NBLK * 128

SCORE_BLK = 8000


def _score_body(h_ref, w_ref, b_ref, uk_ref):
    hb = h_ref[...]
    acc = lax.dot_general(hb, w_ref[...], (((1,), (0,)), ((), ())))
    x = acc[:, 0] + b_ref[0, 0]
    s = 1.0 / (1.0 + jnp.exp(-x))
    uk = KMAX - lax.bitcast_convert_type(s, jnp.int32)
    uk_ref[...] = uk.reshape(1, 8, SCORE_BLK // 8)


def _score_keys(h2, W, b):
    grid = (h2.shape[0] // SCORE_BLK,)
    wmat = jnp.broadcast_to(W[0][:, None], (F, F))
    uk = pl.pallas_call(
        _score_body,
        grid=grid,
        in_specs=[
            pl.BlockSpec((SCORE_BLK, F), lambda j: (j, 0)),
            pl.BlockSpec((F, F), lambda j: (0, 0)),
            pl.BlockSpec((1, 1), lambda j: (0, 0)),
        ],
        out_specs=pl.BlockSpec((1, 8, SCORE_BLK // 8), lambda j: (j, 0, 0)),
        out_shape=jax.ShapeDtypeStruct((grid[0], 8, SCORE_BLK // 8), jnp.int32),
    )(h2, wmat, b.reshape(1, 1))
    return uk.reshape(NB, N)


def _permute(vec, idx):
    dn = lax.GatherDimensionNumbers(
        offset_dims=(), collapsed_slice_dims=(0,), start_index_map=(0,))
    return lax.gather(vec, idx[:, None], dn, (1,),
                      mode=lax.GatherScatterMode.PROMISE_IN_BOUNDS)


def _sc_sort_gather(ukey_pad, h2):
    mesh = plsc.VectorSubcoreMesh(
        core_axis_name="c", subcore_axis_name="s", num_cores=2, num_subcores=T)

    @functools.partial(
        pl.kernel,
        mesh=mesh,
        compiler_params=pltpu.CompilerParams(needs_layout_passes=False),
        out_type=[
            jax.ShapeDtypeStruct((NB, KEEP, F), jnp.float32),
        ],
        scratch_types=[
            pltpu.VMEM_SHARED((NP,), jnp.int32),
            pltpu.VMEM_SHARED((NP,), jnp.int32),
            pltpu.VMEM_SHARED((NP,), jnp.int32),
            pltpu.VMEM_SHARED((NP,), jnp.int32),
            pltpu.VMEM_SHARED((T, RADIX), jnp.int32),
            pltpu.VMEM((C,), jnp.int32),
            pltpu.VMEM((C,), jnp.int32),
            pltpu.VMEM((C,), jnp.int32),
            pltpu.VMEM((C // 128, 128), jnp.int32),
            pltpu.VMEM((RADIX,), jnp.int32),
            pltpu.VMEM((T, RADIX), jnp.int32),
            pltpu.VMEM((16,), jnp.int32),
            pltpu.VMEM((2, 128), jnp.int32),
            pltpu.VMEM((2, 128), jnp.int32),
            pltpu.VMEM((128,), jnp.float32),
            pltpu.VMEM((2, 128, F), jnp.float32),
            pltpu.SemaphoreType.DMA,
            pltpu.SemaphoreType.DMA,
            pltpu.SemaphoreType.DMA,
            pltpu.SemaphoreType.DMA,
            pltpu.SemaphoreType.DMA,
        ],
    )
    def body(ukey_hbm, h2_hbm, o_hbm,
             A_k, A_v, B_k, B_v, Hgrid,
             ck, cv, rinfo, dest, cnt, hl, tmp16, idxbuf, keybuf, sbuf, rows,
             sem, gsem0, gsem1, wsem0, wsem1):
        core = lax.axis_index("c")
        wid = lax.axis_index("s")
        lane = lax.iota(jnp.int32, 16)
        zeros16 = jnp.zeros((16,), jnp.int32)
        base = wid * C

        def vreg_rank(k, shift):
            d = lax.shift_right_logical(k, shift) & (RADIX - 1)
            key2 = d * 16 + lane
            sk, _ = plsc.sort_key_val(key2, key2)
            sd = lax.shift_right_logical(sk, 4)
            sl = sk & 15
            prev = _permute(sd, jnp.maximum(lane - 1, 0))
            nxt = _permute(sd, jnp.minimum(lane + 1, 15))
            is_new = (lane == 0) | (sd != prev)
            is_last = (lane == 15) | (sd != nxt)
            runstart = plsc.cummax(jnp.where(is_new, lane, zeros16))
            r = lane - runstart
            return sd, sl, r, is_last

        for bi in range(2):
            b = core * 2 + bi
            rowbase = b * N

            for p, shift in enumerate((0, 10, 20)):
                src_k = (None, A_k, B_k)[p]
                src_v = (None, A_v, B_v)[p]
                dst_k = (A_k, B_k, A_k)[p]
                dst_v = (A_v, B_v, A_v)[p]

                if p == 0:
                    pltpu.sync_copy(ukey_hbm.at[b, pl.ds(base, C)], ck)
                else:
                    pltpu.sync_copy(src_k.at[pl.ds(base, C)], ck)
                for j in range(RADIX // 16):
                    cnt[pl.ds(j * 16, 16)] = zeros16

                def hist_body(i, carry):
                    k = ck[pl.ds(i * 16, 16)]
                    sd, sl, r, is_last = vreg_rank(k, shift)
                    plsc.addupdate_scatter(cnt, [sd], r + 1, mask=is_last)
                    rinfo[pl.ds(i * 16, 16)] = (
                        sd | (sl << 10) | (r << 14)
                        | (is_last.astype(jnp.int32) << 18))
                    return carry
                lax.fori_loop(0, NV, hist_body, 0)
                pltpu.sync_copy(cnt, Hgrid.at[wid])
                plsc.subcore_barrier()

                pltpu.sync_copy(Hgrid, hl)

                def off_body(j, carry):
                    tot = zeros16
                    mine = zeros16
                    for w in range(T):
                        hv = hl[w, pl.ds(j * 16, 16)]
                        tot = tot + hv
                        mine = mine + jnp.where(w < wid, hv, zeros16)
                    inc = jnp.cumsum(tot)
                    excl = inc - tot
                    cnt[pl.ds(j * 16, 16)] = excl + mine + carry
                    return carry + lax.reduce_sum(tot, axes=(0,))
                lax.fori_loop(0, RADIX // 16, off_body, jnp.int32(0))

                if p > 0:
                    pltpu.sync_copy(src_v.at[pl.ds(base, C)], cv)

                def rank_body(i, carry):
                    pk = rinfo[pl.ds(i * 16, 16)]
                    sd = pk & (RADIX - 1)
                    sl = lax.shift_right_logical(pk, 10) & 15
                    r = lax.shift_right_logical(pk, 14) & 15
                    is_last = (lax.shift_right_logical(pk, 18) & 1) == 1
                    off = plsc.load_gather(cnt, [sd])
                    plsc.addupdate_scatter(cnt, [sd], r + 1, mask=is_last)
                    dsort = off + r
                    plsc.store_scatter(tmp16, [sl], dsort)
                    dorig = tmp16[...]
                    row = lax.shift_right_logical(i, 3)
                    col = (i & 7) * 16
                    plsc.store_scatter(
                        dest, [jnp.broadcast_to(row, (16,)), col + lane], dorig)
                    if p == 0:
                        cv[pl.ds(i * 16, 16)] = rowbase + base + i * 16 + lane
                    return carry
                lax.fori_loop(0, NV, rank_body, 0)

                def scat_body(jg, carry):
                    handles = []
                    for u in range(7):
                        j = jg * 7 + u
                        handles.append(pltpu.async_copy(
                            ck.at[pl.ds(j * 128, 128)],
                            dst_k.at[dest.at[j]], sem))
                        handles.append(pltpu.async_copy(
                            cv.at[pl.ds(j * 128, 128)],
                            dst_v.at[dest.at[j]], sem))
                    for hdl in handles:
                        hdl.wait()
                    return carry
                lax.fori_loop(0, C // 128 // 7, scat_body, 0)
                plsc.subcore_barrier()

            def _start(u, j):
                iv, kv, rv = idxbuf.at[u], keybuf.at[u], rows.at[u]
                gs = (gsem0, gsem1)[u]
                pltpu.sync_copy(A_v.at[pl.ds(j * 128, 128)], iv)
                pltpu.sync_copy(A_k.at[pl.ds(j * 128, 128)], kv)
                pltpu.async_copy(h2_hbm.at[iv], rv, gs)

            def _consume(u, j):
                iv, kv, rv = idxbuf.at[u], keybuf.at[u], rows.at[u]
                gs = (gsem0, gsem1)[u]
                ws = (wsem0, wsem1)[u]
                pltpu.make_async_copy(h2_hbm.at[iv], rv, gs).wait()
                for g in range(8):
                    k16 = kv[pl.ds(g * 16, 16)]
                    sbuf[pl.ds(g * 16, 16)] = lax.bitcast_convert_type(
                        KMAX - k16, jnp.float32)

                def mul_body(t4, carry):
                    r0 = t4 * 4
                    sv = sbuf[pl.ds((r0 // 16) * 16, 16)]
                    for u in range(4):
                        r = r0 + u
                        sb = _permute(sv, jnp.broadcast_to(r % 16, (16,)))
                        for cg in range(8):
                            rv[r, pl.ds(cg * 16, 16)] = (
                                rv[r, pl.ds(cg * 16, 16)] * sb)
                    return carry
                lax.fori_loop(0, 32, mul_body, 0)

                @pl.when(j >= 2 * T)
                def _():
                    pltpu.make_async_copy(
                        rv, o_hbm.at[b, pl.ds((j - 2 * T) * 128, 128)],
                        ws).wait()

                @pl.when(j < NBLK - 1)
                def _():
                    pltpu.async_copy(rv, o_hbm.at[b, pl.ds(j * 128, 128)], ws)

                @pl.when(j == NBLK - 1)
                def _():
                    pltpu.sync_copy(
                        rv.at[pl.ds(0, KEEP - (NBLK - 1) * 128)],
                        o_hbm.at[b, pl.ds((NBLK - 1) * 128,
                                          KEEP - (NBLK - 1) * 128)])

            def gath_body(t, carry):
                j = wid + t * T
                jprev = j - T
                even = (t & 1) == 0

                @pl.when(j < NBLK)
                def _():
                    @pl.when(even)
                    def _():
                        _start(0, j)

                    @pl.when(jnp.logical_not(even))
                    def _():
                        _start(1, j)

                @pl.when((t >= 1) & (jprev < NBLK))
                def _():
                    @pl.when(even)
                    def _():
                        _consume(1, jprev)

                    @pl.when(jnp.logical_not(even))
                    def _():
                        _consume(0, jprev)
                return carry
            lax.fori_loop(0, (NBLK + T - 1) // T + 1, gath_body, 0)
            for u in range(2):
                m = wid + u * T
                jlast = m + ((NBLK - 1 - m) // (2 * T)) * (2 * T)

                @pl.when(jlast != NBLK - 1)
                def _():
                    pltpu.make_async_copy(
                        rows.at[u], o_hbm.at[b, pl.ds(jlast * 128, 128)],
                        (wsem0, wsem1)[u]).wait()
            plsc.subcore_barrier()

    return body(ukey_pad, h2)


def kernel(h, W, b):
    h2 = h.reshape(NB * N, F)
    ukey = _score_keys(h2, W, b)
    ukey_pad = jnp.concatenate(
        [ukey, jnp.full((NB, NP - N), KMAX, jnp.int32)], axis=1)
    (out,) = _sc_sort_gather(ukey_pad, h2)
    return out


# speedup vs baseline: 8.9402x; 1.0305x over previous
import functools

import jax
import jax.numpy as jnp
from jax import lax
from jax.experimental import pallas as pl
from jax.experimental.pallas import tpu as pltpu
from jax.experimental.pallas import tpu_sc as plsc

F = 128
NB = 4
N = 100000
KEEP = 50000
NP = 100352
T = 16
C = NP // T
NV = C // 16
RADIX = 1024
KMAX = 0x3FFFFFFF
NBLK = (KEEP + 127) // 128
KPAD = NBLK * 128

SCORE_BLK = 20000


def _score_body(h_ref, w_ref, b_ref, uk_ref):
    hb = h_ref[...]
    acc = lax.dot_general(hb, w_ref[...], (((1,), (0,)), ((), ())))
    x = acc[:, 0] + b_ref[0, 0]
    s = 1.0 / (1.0 + jnp.exp(-x))
    uk = KMAX - lax.bitcast_convert_type(s, jnp.int32)
    uk_ref[...] = uk.reshape(1, 8, SCORE_BLK // 8)


def _score_keys(h2, W, b):
    grid = (h2.shape[0] // SCORE_BLK,)
    wmat = jnp.broadcast_to(W[0][:, None], (F, F))
    uk = pl.pallas_call(
        _score_body,
        grid=grid,
        in_specs=[
            pl.BlockSpec((SCORE_BLK, F), lambda j: (j, 0)),
            pl.BlockSpec((F, F), lambda j: (0, 0)),
            pl.BlockSpec((1, 1), lambda j: (0, 0)),
        ],
        out_specs=pl.BlockSpec((1, 8, SCORE_BLK // 8), lambda j: (j, 0, 0)),
        out_shape=jax.ShapeDtypeStruct((grid[0], 8, SCORE_BLK // 8), jnp.int32),
    )(h2, wmat, b.reshape(1, 1))
    return uk.reshape(NB, N)


def _permute(vec, idx):
    dn = lax.GatherDimensionNumbers(
        offset_dims=(), collapsed_slice_dims=(0,), start_index_map=(0,))
    return lax.gather(vec, idx[:, None], dn, (1,),
                      mode=lax.GatherScatterMode.PROMISE_IN_BOUNDS)


def _sc_sort_gather(ukey_pad, h2):
    mesh = plsc.VectorSubcoreMesh(
        core_axis_name="c", subcore_axis_name="s", num_cores=2, num_subcores=T)

    @functools.partial(
        pl.kernel,
        mesh=mesh,
        compiler_params=pltpu.CompilerParams(needs_layout_passes=False),
        out_type=[
            jax.ShapeDtypeStruct((NB, KEEP, F), jnp.float32),
        ],
        scratch_types=[
            pltpu.VMEM_SHARED((NP,), jnp.int32),
            pltpu.VMEM_SHARED((NP,), jnp.int32),
            pltpu.VMEM_SHARED((NP,), jnp.int32),
            pltpu.VMEM_SHARED((NP,), jnp.int32),
            pltpu.VMEM_SHARED((T, RADIX), jnp.int32),
            pltpu.VMEM((C,), jnp.int32),
            pltpu.VMEM((C,), jnp.int32),
            pltpu.VMEM((C,), jnp.int32),
            pltpu.VMEM((C // 128, 128), jnp.int32),
            pltpu.VMEM((RADIX,), jnp.int32),
            pltpu.VMEM((T, RADIX), jnp.int32),
            pltpu.VMEM((16,), jnp.int32),
            pltpu.VMEM((2, 128), jnp.int32),
            pltpu.VMEM((2, 128), jnp.int32),
            pltpu.VMEM((128,), jnp.float32),
            pltpu.VMEM((2, 128, F), jnp.float32),
            pltpu.SemaphoreType.DMA,
            pltpu.SemaphoreType.DMA,
            pltpu.SemaphoreType.DMA,
            pltpu.SemaphoreType.DMA,
            pltpu.SemaphoreType.DMA,
        ],
    )
    def body(ukey_hbm, h2_hbm, o_hbm,
             A_k, A_v, B_k, B_v, Hgrid,
             ck, cv, rinfo, dest, cnt, hl, tmp16, idxbuf, keybuf, sbuf, rows,
             sem, gsem0, gsem1, wsem0, wsem1):
        core = lax.axis_index("c")
        wid = lax.axis_index("s")
        lane = lax.iota(jnp.int32, 16)
        zeros16 = jnp.zeros((16,), jnp.int32)
        base = wid * C

        def vreg_rank(k, shift):
            d = lax.shift_right_logical(k, shift) & (RADIX - 1)
            key2 = d * 16 + lane
            sk, _ = plsc.sort_key_val(key2, key2)
            sd = lax.shift_right_logical(sk, 4)
            sl = sk & 15
            prev = _permute(sd, jnp.maximum(lane - 1, 0))
            nxt = _permute(sd, jnp.minimum(lane + 1, 15))
            is_new = (lane == 0) | (sd != prev)
            is_last = (lane == 15) | (sd != nxt)
            runstart = plsc.cummax(jnp.where(is_new, lane, zeros16))
            r = lane - runstart
            return sd, sl, r, is_last

        for bi in range(2):
            b = core * 2 + bi
            rowbase = b * N

            for p, shift in enumerate((0, 10, 20)):
                src_k = (None, A_k, B_k)[p]
                src_v = (None, A_v, B_v)[p]
                dst_k = (A_k, B_k, A_k)[p]
                dst_v = (A_v, B_v, A_v)[p]

                if p == 0:
                    pltpu.sync_copy(ukey_hbm.at[b, pl.ds(base, C)], ck)
                else:
                    pltpu.sync_copy(src_k.at[pl.ds(base, C)], ck)
                for j in range(RADIX // 16):
                    cnt[pl.ds(j * 16, 16)] = zeros16

                def hist_body(i, carry):
                    k = ck[pl.ds(i * 16, 16)]
                    sd, sl, r, is_last = vreg_rank(k, shift)
                    plsc.addupdate_scatter(cnt, [sd], r + 1, mask=is_last)
                    rinfo[pl.ds(i * 16, 16)] = (
                        sd | (sl << 10) | (r << 14)
                        | (is_last.astype(jnp.int32) << 18))
                    return carry
                lax.fori_loop(0, NV, hist_body, 0)
                pltpu.sync_copy(cnt, Hgrid.at[wid])
                plsc.subcore_barrier()

                pltpu.sync_copy(Hgrid, hl)

                def off_body(j, carry):
                    tot = zeros16
                    mine = zeros16
                    for w in range(T):
                        hv = hl[w, pl.ds(j * 16, 16)]
                        tot = tot + hv
                        mine = mine + jnp.where(w < wid, hv, zeros16)
                    inc = jnp.cumsum(tot)
                    excl = inc - tot
                    cnt[pl.ds(j * 16, 16)] = excl + mine + carry
                    return carry + lax.reduce_sum(tot, axes=(0,))
                lax.fori_loop(0, RADIX // 16, off_body, jnp.int32(0))

                if p > 0:
                    pltpu.sync_copy(src_v.at[pl.ds(base, C)], cv)

                def rank_body(i, carry):
                    pk = rinfo[pl.ds(i * 16, 16)]
                    sd = pk & (RADIX - 1)
                    sl = lax.shift_right_logical(pk, 10) & 15
                    r = lax.shift_right_logical(pk, 14) & 15
                    is_last = (lax.shift_right_logical(pk, 18) & 1) == 1
                    off = plsc.load_gather(cnt, [sd])
                    plsc.addupdate_scatter(cnt, [sd], r + 1, mask=is_last)
                    dsort = off + r
                    plsc.store_scatter(tmp16, [sl], dsort)
                    dorig = tmp16[...]
                    row = lax.shift_right_logical(i, 3)
                    col = (i & 7) * 16
                    plsc.store_scatter(
                        dest, [jnp.broadcast_to(row, (16,)), col + lane], dorig)
                    if p == 0:
                        cv[pl.ds(i * 16, 16)] = rowbase + base + i * 16 + lane
                    return carry
                lax.fori_loop(0, NV, rank_body, 0)

                def scat_body(jg, carry):
                    handles = []
                    for u in range(7):
                        j = jg * 7 + u
                        handles.append(pltpu.async_copy(
                            ck.at[pl.ds(j * 128, 128)],
                            dst_k.at[dest.at[j]], sem))
                        handles.append(pltpu.async_copy(
                            cv.at[pl.ds(j * 128, 128)],
                            dst_v.at[dest.at[j]], sem))
                    for hdl in handles:
                        hdl.wait()
                    return carry
                lax.fori_loop(0, C // 128 // 7, scat_body, 0)
                plsc.subcore_barrier()

            def _start(u, j):
                iv, kv, rv = idxbuf.at[u], keybuf.at[u], rows.at[u]
                gs = (gsem0, gsem1)[u]
                pltpu.sync_copy(A_v.at[pl.ds(j * 128, 128)], iv)
                pltpu.sync_copy(A_k.at[pl.ds(j * 128, 128)], kv)
                pltpu.async_copy(h2_hbm.at[iv], rv, gs)

            def _consume(u, j):
                iv, kv, rv = idxbuf.at[u], keybuf.at[u], rows.at[u]
                gs = (gsem0, gsem1)[u]
                ws = (wsem0, wsem1)[u]
                pltpu.make_async_copy(h2_hbm.at[iv], rv, gs).wait()
                for g in range(8):
                    k16 = kv[pl.ds(g * 16, 16)]
                    sbuf[pl.ds(g * 16, 16)] = lax.bitcast_convert_type(
                        KMAX - k16, jnp.float32)

                def mul_body(t4, carry):
                    r0 = t4 * 4
                    sv = sbuf[pl.ds((r0 // 16) * 16, 16)]
                    for u in range(4):
                        r = r0 + u
                        sb = _permute(sv, jnp.broadcast_to(r % 16, (16,)))
                        for cg in range(8):
                            rv[r, pl.ds(cg * 16, 16)] = (
                                rv[r, pl.ds(cg * 16, 16)] * sb)
                    return carry
                lax.fori_loop(0, 32, mul_body, 0)

                @pl.when(j >= 2 * T)
                def _():
                    pltpu.make_async_copy(
                        rv, o_hbm.at[b, pl.ds((j - 2 * T) * 128, 128)],
                        ws).wait()

                @pl.when(j < NBLK - 1)
                def _():
                    pltpu.async_copy(rv, o_hbm.at[b, pl.ds(j * 128, 128)], ws)

                @pl.when(j == NBLK - 1)
                def _():
                    pltpu.sync_copy(
                        rv.at[pl.ds(0, KEEP - (NBLK - 1) * 128)],
                        o_hbm.at[b, pl.ds((NBLK - 1) * 128,
                                          KEEP - (NBLK - 1) * 128)])

            def gath_body(t, carry):
                j = wid + t * T
                jprev = j - T
                even = (t & 1) == 0

                @pl.when(j < NBLK)
                def _():
                    @pl.when(even)
                    def _():
                        _start(0, j)

                    @pl.when(jnp.logical_not(even))
                    def _():
                        _start(1, j)

                @pl.when((t >= 1) & (jprev < NBLK))
                def _():
                    @pl.when(even)
                    def _():
                        _consume(1, jprev)

                    @pl.when(jnp.logical_not(even))
                    def _():
                        _consume(0, jprev)
                return carry
            lax.fori_loop(0, (NBLK + T - 1) // T + 1, gath_body, 0)
            for u in range(2):
                m = wid + u * T
                jlast = m + ((NBLK - 1 - m) // (2 * T)) * (2 * T)

                @pl.when(jlast != NBLK - 1)
                def _():
                    pltpu.make_async_copy(
                        rows.at[u], o_hbm.at[b, pl.ds(jlast * 128, 128)],
                        (wsem0, wsem1)[u]).wait()
            plsc.subcore_barrier()

    return body(ukey_pad, h2)


def kernel(h, W, b):
    h2 = h.reshape(NB * N, F)
    ukey = _score_keys(h2, W, b)
    ukey_pad = jnp.concatenate(
        [ukey, jnp.full((NB, NP - N), KMAX, jnp.int32)], axis=1)
    (out,) = _sc_sort_gather(ukey_pad, h2)
    return out
